# Initial kernel scaffold; baseline (speedup 1.0000x reference)
import jax
import jax.numpy as jnp
from jax.experimental import pallas as pl


def kernel(z_e, e):
    raise NotImplementedError("write your pallas kernel here")


# speedup vs baseline: 2.0793x; 2.0793x over previous
import functools

import jax
import jax.numpy as jnp
from jax.experimental import pallas as pl

_NE = 512
_D = 64
_BB = 8


def _tc_body(z_ref, e_ref, out_ref):
    e_mat = e_ref[...]
    s = jnp.sum(e_mat * e_mat, axis=1, keepdims=True)
    jid = jax.lax.broadcasted_iota(jnp.int32, (_NE, z_ref.shape[2]), 0)
    for b in range(z_ref.shape[0]):
        zb = z_ref[b]
        m = jax.lax.dot_general(
            e_mat, zb, (((1,), (0,)), ((), ())),
            preferred_element_type=jnp.float32,
            precision=jax.lax.Precision.DEFAULT)
        d = s - 2.0 * m
        dmin = jnp.min(d, axis=0, keepdims=True)
        cand = jnp.where(d == dmin, jid, jnp.int32(_NE))
        idx = jnp.min(cand, axis=0, keepdims=True)
        onehot = (jid == idx).astype(jnp.float32)
        zq = jax.lax.dot_general(
            e_mat, onehot, (((0,), (0,)), ((), ())),
            preferred_element_type=jnp.float32,
            precision=jax.lax.Precision.HIGHEST)
        out_ref[b] = zq


@functools.partial(jax.jit, static_argnums=())
def kernel(z_e, e):
    B, C, H, W = z_e.shape
    P = H * W
    z3 = z_e.reshape(B, C, P)
    grid = (B // _BB,)
    out = pl.pallas_call(
        _tc_body,
        grid=grid,
        in_specs=[
            pl.BlockSpec((_BB, C, P), lambda i: (i, 0, 0)),
            pl.BlockSpec((_NE, _D), lambda i: (0, 0)),
        ],
        out_specs=pl.BlockSpec((_BB, C, P), lambda i: (i, 0, 0)),
        out_shape=jax.ShapeDtypeStruct((B, C, P), jnp.float32),
    )(z3, e)
    return out.reshape(B, C, H, W)
